# baseline (speedup 1.0000x reference)
import functools

import jax
import jax.numpy as jnp
from jax import lax
from jax.experimental import pallas as pl
from jax.experimental.pallas import tpu as pltpu
from jax.experimental.pallas import tpu_sc as plsc

L = 16
NC = 2
NS = 16
NW = NC * NS


@functools.lru_cache(maxsize=None)
def _make_sc_kernel(B, S, P, C):
    S2 = S * S
    npts = (B * P) // NW
    nch = npts // C
    assert npts % C == 0 and C % L == 0
    assert P % npts == 0

    mesh = plsc.VectorSubcoreMesh(core_axis_name="c", subcore_axis_name="s")

    @functools.partial(
        pl.kernel,
        mesh=mesh,
        out_type=jax.ShapeDtypeStruct((NW, 2, L), jnp.float32),
        scratch_types=[
            pltpu.VMEM((C,), jnp.int32),
            pltpu.VMEM((C,), jnp.int32),
            pltpu.VMEM((C,), jnp.int32),
            pltpu.VMEM((C,), jnp.float32),
            pltpu.VMEM((C,), jnp.float32),
            pltpu.VMEM((L,), jnp.float32),
            pltpu.SemaphoreType.DMA,
            pltpu.SemaphoreType.DMA,
        ],
    )
    def sc_kernel(in_hbm, w_hbm, sx_hbm, sy_hbm, dx_hbm, dy_hbm, out_hbm,
                  x_v, y_v, idx_v, g_v, gw_v, res_v, sem1, sem2):
        wid = lax.axis_index("s") * NC + lax.axis_index("c")
        start = wid * npts
        batch_base = (start // P) * S2

        def process(xi_hbm, yi_hbm, is_same):
            def chunk_body(ci, acc):
                off = start + ci * C
                pltpu.sync_copy(xi_hbm.at[pl.ds(off, C)], x_v)
                pltpu.sync_copy(yi_hbm.at[pl.ds(off, C)], y_v)

                def flat_body(j, carry):
                    sl = pl.ds(j * L, L)
                    idx_v[sl] = x_v[sl] * S + y_v[sl] + batch_base
                    return carry
                lax.fori_loop(0, C // L, flat_body, 0, unroll=4)

                cp1 = pltpu.async_copy(in_hbm.at[idx_v], g_v, sem1)
                cp2 = pltpu.async_copy(w_hbm.at[idx_v], gw_v, sem2)
                cp1.wait()
                cp2.wait()

                def acc_body(j, a):
                    sl = pl.ds(j * L, L)
                    vi = g_v[sl]
                    vw = gw_v[sl]
                    return a + (vi * vw if is_same else vi / vw)
                return lax.fori_loop(0, C // L, acc_body, acc, unroll=4)

            return lax.fori_loop(0, nch, chunk_body,
                                 jnp.zeros((L,), jnp.float32))

        acc_s = process(sx_hbm, sy_hbm, True)
        acc_d = process(dx_hbm, dy_hbm, False)
        res_v[...] = acc_s
        pltpu.sync_copy(res_v, out_hbm.at[wid, 0])
        res_v[...] = acc_d
        pltpu.sync_copy(res_v, out_hbm.at[wid, 1])

    return sc_kernel


def kernel(input, weights, same_x, same_y, diff_x, diff_y):
    B, S, _ = input.shape
    P = same_x.shape[1]

    in_flat = input.reshape(B * S * S)
    w_flat = weights.reshape(B * S * S)
    sx = same_x.astype(jnp.int32).reshape(-1)
    sy = same_y.astype(jnp.int32).reshape(-1)
    dx = diff_x.astype(jnp.int32).reshape(-1)
    dy = diff_y.astype(jnp.int32).reshape(-1)

    out = _make_sc_kernel(B, S, P, 4096)(in_flat, w_flat, sx, sy, dx, dy)
    n = jnp.float32(B * P)
    return out[:, 0, :].sum() / n - out[:, 1, :].sum() / n


# speedup vs baseline: 1.0990x; 1.0990x over previous
import functools

import jax
import jax.numpy as jnp
from jax import lax
from jax.experimental import pallas as pl
from jax.experimental.pallas import tpu as pltpu
from jax.experimental.pallas import tpu_sc as plsc

L = 16
NC = 2
NS = 16
NW = NC * NS


@functools.lru_cache(maxsize=None)
def _make_sc_kernel(B, S, P, C):
    S2 = S * S
    BP = B * P
    npts = BP // NW
    nch = npts // C
    T = 2 * nch
    assert npts % C == 0 and C % L == 0 and nch % 2 == 0
    assert P % npts == 0

    mesh = plsc.VectorSubcoreMesh(core_axis_name="c", subcore_axis_name="s")

    @functools.partial(
        pl.kernel,
        mesh=mesh,
        out_type=jax.ShapeDtypeStruct((NW, 2, L), jnp.float32),
        scratch_types=[
            pltpu.VMEM((C,), jnp.int32),
            pltpu.VMEM((C,), jnp.int32),
            pltpu.VMEM((C,), jnp.int32),
            pltpu.VMEM((C,), jnp.int32),
            pltpu.VMEM((C,), jnp.int32),
            pltpu.VMEM((C,), jnp.int32),
            pltpu.VMEM((C,), jnp.float32),
            pltpu.VMEM((C,), jnp.float32),
            pltpu.VMEM((C,), jnp.float32),
            pltpu.VMEM((C,), jnp.float32),
            pltpu.VMEM((L,), jnp.float32),
            pltpu.VMEM((L,), jnp.float32),
            pltpu.VMEM((L,), jnp.float32),
            pltpu.SemaphoreType.DMA,
            pltpu.SemaphoreType.DMA,
            pltpu.SemaphoreType.DMA,
            pltpu.SemaphoreType.DMA,
        ],
    )
    def sc_kernel(in_hbm, w_hbm, xs_hbm, ys_hbm, out_hbm,
                  x0, y0, x1, y1, idx0, idx1, gi0, gw0, gi1, gw1,
                  res_v, acc_s_v, acc_d_v, si0, si1, sg0, sg1):
        wid = lax.axis_index("s") * NC + lax.axis_index("c")
        start = wid * npts
        batch_base = (start // P) * S2

        def off_of(c):
            return start + c * C + jnp.where(c >= nch, BP - npts, 0)

        def fire_xy(c, xv, yv, sem):
            o = off_of(c)
            pltpu.async_copy(xs_hbm.at[pl.ds(o, C)], xv, sem)
            pltpu.async_copy(ys_hbm.at[pl.ds(o, C)], yv, sem)

        def wait_xy(c, xv, yv, sem):
            o = off_of(c)
            pltpu.make_async_copy(xs_hbm.at[pl.ds(o, C)], xv, sem).wait()
            pltpu.make_async_copy(ys_hbm.at[pl.ds(o, C)], yv, sem).wait()

        def flat(xv, yv, idxv):
            def body(j, carry):
                sl = pl.ds(j * L, L)
                idxv[sl] = xv[sl] * S + yv[sl] + batch_base
                return carry
            lax.fori_loop(0, C // L, body, 0, unroll=4)

        def fire_gather(idxv, giv, gwv, sem):
            pltpu.async_copy(in_hbm.at[idxv], giv, sem)
            pltpu.async_copy(w_hbm.at[idxv], gwv, sem)

        def wait_gather(idxv, giv, gwv, sem):
            pltpu.make_async_copy(in_hbm.at[idxv], giv, sem).wait()
            pltpu.make_async_copy(w_hbm.at[idxv], gwv, sem).wait()

        def accumulate(c, giv, gwv):
            @pl.when(c < nch)
            def _():
                def body(j, aa):
                    sl = pl.ds(j * L, L)
                    return aa + giv[sl] * gwv[sl]
                acc_s_v[...] = lax.fori_loop(0, C // L, body, acc_s_v[...],
                                             unroll=4)

            @pl.when(c >= nch)
            def _():
                def body(j, aa):
                    sl = pl.ds(j * L, L)
                    return aa + giv[sl] / gwv[sl]
                acc_d_v[...] = lax.fori_loop(0, C // L, body, acc_d_v[...],
                                             unroll=4)

        acc_s_v[...] = jnp.zeros((L,), jnp.float32)
        acc_d_v[...] = jnp.zeros((L,), jnp.float32)
        fire_xy(0, x0, y0, si0)
        wait_xy(0, x0, y0, si0)
        flat(x0, y0, idx0)
        fire_gather(idx0, gi0, gw0, sg0)
        fire_xy(1, x1, y1, si1)

        def outer(k, carry):
            c0 = 2 * k
            c1 = c0 + 1
            wait_xy(c1, x1, y1, si1)
            flat(x1, y1, idx1)
            fire_gather(idx1, gi1, gw1, sg1)

            @pl.when(c0 + 2 < T)
            def _():
                fire_xy(c0 + 2, x0, y0, si0)

            wait_gather(idx0, gi0, gw0, sg0)
            accumulate(c0, gi0, gw0)

            @pl.when(c0 + 2 < T)
            def _():
                wait_xy(c0 + 2, x0, y0, si0)
                flat(x0, y0, idx0)
                fire_gather(idx0, gi0, gw0, sg0)

            @pl.when(c1 + 2 < T)
            def _():
                fire_xy(c1 + 2, x1, y1, si1)

            wait_gather(idx1, gi1, gw1, sg1)
            accumulate(c1, gi1, gw1)
            return carry

        lax.fori_loop(0, T // 2, outer, 0)

        res_v[...] = acc_s_v[...]
        pltpu.sync_copy(res_v, out_hbm.at[wid, 0])
        res_v[...] = acc_d_v[...]
        pltpu.sync_copy(res_v, out_hbm.at[wid, 1])

    return sc_kernel


def kernel(input, weights, same_x, same_y, diff_x, diff_y):
    B, S, _ = input.shape
    P = same_x.shape[1]

    in_flat = input.reshape(B * S * S)
    w_flat = weights.reshape(B * S * S)
    xs = jnp.concatenate([same_x.astype(jnp.int32).reshape(-1),
                          diff_x.astype(jnp.int32).reshape(-1)])
    ys = jnp.concatenate([same_y.astype(jnp.int32).reshape(-1),
                          diff_y.astype(jnp.int32).reshape(-1)])

    out = _make_sc_kernel(B, S, P, 8192)(in_flat, w_flat, xs, ys)
    n = jnp.float32(B * P)
    return out[:, 0, :].sum() / n - out[:, 1, :].sum() / n


# speedup vs baseline: 1.2255x; 1.1151x over previous
import functools

import jax
import jax.numpy as jnp
from jax import lax
from jax.experimental import pallas as pl
from jax.experimental.pallas import tpu as pltpu
from jax.experimental.pallas import tpu_sc as plsc

L = 16
NC = 2
NS = 16
NW = NC * NS


@functools.lru_cache(maxsize=None)
def _make_sc_kernel(B, S, P, C):
    S2 = S * S
    npts = (B * P) // NW
    nch = npts // C
    T = 2 * nch
    WPB = P // npts
    assert npts % C == 0 and C % L == 0 and nch % 2 == 0
    assert P % npts == 0

    mesh = plsc.VectorSubcoreMesh(core_axis_name="c", subcore_axis_name="s")

    @functools.partial(
        pl.kernel,
        mesh=mesh,
        out_type=jax.ShapeDtypeStruct((NW, 2, L), jnp.float32),
        scratch_types=[
            pltpu.VMEM((C,), jnp.int32),
            pltpu.VMEM((C,), jnp.int32),
            pltpu.VMEM((C,), jnp.int32),
            pltpu.VMEM((C,), jnp.int32),
            pltpu.VMEM((C,), jnp.int32),
            pltpu.VMEM((C,), jnp.int32),
            pltpu.VMEM((C,), jnp.float32),
            pltpu.VMEM((C,), jnp.float32),
            pltpu.VMEM((C,), jnp.float32),
            pltpu.VMEM((C,), jnp.float32),
            pltpu.VMEM((L,), jnp.float32),
            pltpu.VMEM((L,), jnp.float32),
            pltpu.VMEM((L,), jnp.float32),
            pltpu.SemaphoreType.DMA,
            pltpu.SemaphoreType.DMA,
            pltpu.SemaphoreType.DMA,
            pltpu.SemaphoreType.DMA,
        ],
    )
    def sc_kernel(in_hbm, w_hbm, sx_hbm, sy_hbm, dx_hbm, dy_hbm, out_hbm,
                  x0, y0, x1, y1, idx0, idx1, gi0, gw0, gi1, gw1,
                  res_v, acc_s_v, acc_d_v, si0, si1, sg0, sg1):
        wid = lax.axis_index("s") * NC + lax.axis_index("c")
        b = wid // WPB
        colbase = (wid % WPB) * npts
        batch_base = b * S2

        def fire_xy(c, xv, yv, sem):
            @pl.when(c < nch)
            def _():
                col = colbase + c * C
                pltpu.async_copy(sx_hbm.at[b, pl.ds(col, C)], xv, sem)
                pltpu.async_copy(sy_hbm.at[b, pl.ds(col, C)], yv, sem)

            @pl.when(c >= nch)
            def _():
                col = colbase + (c - nch) * C
                pltpu.async_copy(dx_hbm.at[b, pl.ds(col, C)], xv, sem)
                pltpu.async_copy(dy_hbm.at[b, pl.ds(col, C)], yv, sem)

        def wait_xy(xv, yv, sem):
            pltpu.make_async_copy(sx_hbm.at[0, pl.ds(0, C)], xv, sem).wait()
            pltpu.make_async_copy(sy_hbm.at[0, pl.ds(0, C)], yv, sem).wait()

        def flat(xv, yv, idxv):
            def body(j, carry):
                sl = pl.ds(j * L, L)
                idxv[sl] = xv[sl] * S + yv[sl] + batch_base
                return carry
            lax.fori_loop(0, C // L, body, 0, unroll=4)

        def fire_gather(idxv, giv, gwv, sem):
            pltpu.async_copy(in_hbm.at[idxv], giv, sem)
            pltpu.async_copy(w_hbm.at[idxv], gwv, sem)

        def wait_gather(idxv, giv, gwv, sem):
            pltpu.make_async_copy(in_hbm.at[idxv], giv, sem).wait()
            pltpu.make_async_copy(w_hbm.at[idxv], gwv, sem).wait()

        def accumulate(c, giv, gwv):
            @pl.when(c < nch)
            def _():
                def body(j, aa):
                    sl = pl.ds(j * L, L)
                    return aa + giv[sl] * gwv[sl]
                acc_s_v[...] = lax.fori_loop(0, C // L, body, acc_s_v[...],
                                             unroll=4)

            @pl.when(c >= nch)
            def _():
                def body(j, aa):
                    sl = pl.ds(j * L, L)
                    return aa + giv[sl] / gwv[sl]
                acc_d_v[...] = lax.fori_loop(0, C // L, body, acc_d_v[...],
                                             unroll=4)

        acc_s_v[...] = jnp.zeros((L,), jnp.float32)
        acc_d_v[...] = jnp.zeros((L,), jnp.float32)
        fire_xy(0, x0, y0, si0)
        wait_xy(x0, y0, si0)
        flat(x0, y0, idx0)
        fire_gather(idx0, gi0, gw0, sg0)
        fire_xy(1, x1, y1, si1)

        def outer(k, carry):
            c0 = 2 * k
            c1 = c0 + 1
            wait_xy(x1, y1, si1)
            flat(x1, y1, idx1)
            fire_gather(idx1, gi1, gw1, sg1)

            @pl.when(c0 + 2 < T)
            def _():
                fire_xy(c0 + 2, x0, y0, si0)

            wait_gather(idx0, gi0, gw0, sg0)
            accumulate(c0, gi0, gw0)

            @pl.when(c0 + 2 < T)
            def _():
                wait_xy(x0, y0, si0)
                flat(x0, y0, idx0)
                fire_gather(idx0, gi0, gw0, sg0)

            @pl.when(c1 + 2 < T)
            def _():
                fire_xy(c1 + 2, x1, y1, si1)

            wait_gather(idx1, gi1, gw1, sg1)
            accumulate(c1, gi1, gw1)
            return carry

        lax.fori_loop(0, T // 2, outer, 0)

        res_v[...] = acc_s_v[...]
        pltpu.sync_copy(res_v, out_hbm.at[wid, 0])
        res_v[...] = acc_d_v[...]
        pltpu.sync_copy(res_v, out_hbm.at[wid, 1])

    return sc_kernel


def kernel(input, weights, same_x, same_y, diff_x, diff_y):
    B, S, _ = input.shape
    P = same_x.shape[1]

    in_flat = input.reshape(B * S * S)
    w_flat = weights.reshape(B * S * S)
    sx = same_x.astype(jnp.int32)
    sy = same_y.astype(jnp.int32)
    dx = diff_x.astype(jnp.int32)
    dy = diff_y.astype(jnp.int32)

    out = _make_sc_kernel(B, S, P, 8192)(in_flat, w_flat, sx, sy, dx, dy)
    n = jnp.float32(B * P)
    return out[:, 0, :].sum() / n - out[:, 1, :].sum() / n


# speedup vs baseline: 1.9245x; 1.5704x over previous
import functools

import jax
import jax.numpy as jnp
from jax import lax
from jax.experimental import pallas as pl
from jax.experimental.pallas import tpu as pltpu
from jax.experimental.pallas import tpu_sc as plsc

L = 16
NC = 2
NS = 16
NW = NC * NS


@functools.lru_cache(maxsize=None)
def _make_sc_kernel(B, S, P, C):
    S2 = S * S
    npts = (B * P) // NW
    nch = npts // C
    T = 2 * nch
    WPB = P // npts
    assert npts % C == 0 and C % L == 0 and nch % 2 == 0
    assert P % npts == 0

    mesh = plsc.VectorSubcoreMesh(core_axis_name="c", subcore_axis_name="s")

    @functools.partial(
        pl.kernel,
        mesh=mesh,
        out_type=jax.ShapeDtypeStruct((NW, 2, L), jnp.float32),
        scratch_types=[
            pltpu.VMEM((C,), jnp.int32),
            pltpu.VMEM((C,), jnp.int32),
            pltpu.VMEM((C,), jnp.int32),
            pltpu.VMEM((C,), jnp.int32),
            pltpu.VMEM((C,), jnp.int32),
            pltpu.VMEM((C,), jnp.int32),
            pltpu.VMEM((C,), jnp.float32),
            pltpu.VMEM((C,), jnp.float32),
            pltpu.VMEM((C,), jnp.float32),
            pltpu.VMEM((C,), jnp.float32),
            pltpu.VMEM((L,), jnp.float32),
            pltpu.VMEM((L,), jnp.float32),
            pltpu.VMEM((L,), jnp.float32),
            pltpu.SemaphoreType.DMA,
            pltpu.SemaphoreType.DMA,
            pltpu.SemaphoreType.DMA,
            pltpu.SemaphoreType.DMA,
        ],
    )
    def sc_kernel(in_hbm, w_hbm, sx_hbm, sy_hbm, dx_hbm, dy_hbm, out_hbm,
                  x0, y0, x1, y1, idx0, idx1, gi0, gw0, gi1, gw1,
                  res_v, acc_s_v, acc_d_v, si0, si1, sg0, sg1):
        wid = lax.axis_index("s") * NC + lax.axis_index("c")
        b = wid // WPB
        colbase = (wid % WPB) * npts
        batch_base = b * S2

        def fire_xy(c, xv, yv, sem):
            @pl.when(c < nch)
            def _():
                col = colbase + c * C
                pltpu.async_copy(sx_hbm.at[b, pl.ds(col, C)], xv, sem)
                pltpu.async_copy(sy_hbm.at[b, pl.ds(col, C)], yv, sem)

            @pl.when(c >= nch)
            def _():
                col = colbase + (c - nch) * C
                pltpu.async_copy(dx_hbm.at[b, pl.ds(col, C)], xv, sem)
                pltpu.async_copy(dy_hbm.at[b, pl.ds(col, C)], yv, sem)

        def wait_xy(xv, yv, sem):
            pltpu.make_async_copy(sx_hbm.at[0, pl.ds(0, C)], xv, sem).wait()
            pltpu.make_async_copy(sy_hbm.at[0, pl.ds(0, C)], yv, sem).wait()

        def flat(xv, yv, idxv):
            def body(j, carry):
                sl = pl.ds(j * L, L)
                r = xv[sl]
                c = yv[sl]
                idxv[sl] = (batch_base
                            + ((r >> 3) << 14) + ((c >> 7) << 10)
                            + ((r & 7) << 7) + (c & 127))
                return carry
            lax.fori_loop(0, C // L, body, 0, unroll=4)

        def fire_gather(idxv, giv, gwv, sem):
            pltpu.async_copy(in_hbm.at[idxv], giv, sem)
            pltpu.async_copy(w_hbm.at[idxv], gwv, sem)

        def wait_gather(idxv, giv, gwv, sem):
            pltpu.make_async_copy(in_hbm.at[idxv], giv, sem).wait()
            pltpu.make_async_copy(w_hbm.at[idxv], gwv, sem).wait()

        def accumulate(c, giv, gwv):
            @pl.when(c < nch)
            def _():
                def body(j, aa):
                    sl = pl.ds(j * L, L)
                    return aa + giv[sl] * gwv[sl]
                acc_s_v[...] = lax.fori_loop(0, C // L, body, acc_s_v[...],
                                             unroll=4)

            @pl.when(c >= nch)
            def _():
                def body(j, aa):
                    sl = pl.ds(j * L, L)
                    return aa + giv[sl] / gwv[sl]
                acc_d_v[...] = lax.fori_loop(0, C // L, body, acc_d_v[...],
                                             unroll=4)

        acc_s_v[...] = jnp.zeros((L,), jnp.float32)
        acc_d_v[...] = jnp.zeros((L,), jnp.float32)
        fire_xy(0, x0, y0, si0)
        wait_xy(x0, y0, si0)
        flat(x0, y0, idx0)
        fire_gather(idx0, gi0, gw0, sg0)
        fire_xy(1, x1, y1, si1)

        def outer(k, carry):
            c0 = 2 * k
            c1 = c0 + 1
            wait_xy(x1, y1, si1)
            flat(x1, y1, idx1)
            fire_gather(idx1, gi1, gw1, sg1)

            @pl.when(c0 + 2 < T)
            def _():
                fire_xy(c0 + 2, x0, y0, si0)

            wait_gather(idx0, gi0, gw0, sg0)
            accumulate(c0, gi0, gw0)

            @pl.when(c0 + 2 < T)
            def _():
                wait_xy(x0, y0, si0)
                flat(x0, y0, idx0)
                fire_gather(idx0, gi0, gw0, sg0)

            @pl.when(c1 + 2 < T)
            def _():
                fire_xy(c1 + 2, x1, y1, si1)

            wait_gather(idx1, gi1, gw1, sg1)
            accumulate(c1, gi1, gw1)
            return carry

        lax.fori_loop(0, T // 2, outer, 0)

        res_v[...] = acc_s_v[...]
        pltpu.sync_copy(res_v, out_hbm.at[wid, 0])
        res_v[...] = acc_d_v[...]
        pltpu.sync_copy(res_v, out_hbm.at[wid, 1])

    return sc_kernel


def kernel(input, weights, same_x, same_y, diff_x, diff_y):
    B, S, _ = input.shape
    P = same_x.shape[1]

    sx = same_x.astype(jnp.int32)
    sy = same_y.astype(jnp.int32)
    dx = diff_x.astype(jnp.int32)
    dy = diff_y.astype(jnp.int32)

    def tiled_view(a):
        a5 = a.reshape(B, S // 8, 8, S // 128, 128)
        return a5.transpose(0, 1, 3, 2, 4).reshape(B * S * S)

    in_flat = tiled_view(input)
    w_flat = tiled_view(weights)
    out = _make_sc_kernel(B, S, P, 8192)(in_flat, w_flat, sx, sy, dx, dy)
    n = jnp.float32(B * P)
    return out[:, 0, :].sum() / n - out[:, 1, :].sum() / n
